# Initial kernel scaffold; baseline (speedup 1.0000x reference)
import jax
import jax.numpy as jnp
from jax.experimental import pallas as pl


def kernel(src_node_ids, dst_node_ids, RP, lambda_weights, W1, b1, W2, b2):
    raise NotImplementedError("write your pallas kernel here")


# speedup vs baseline: 3.3566x; 3.3566x over previous
import functools
import math

import jax
import jax.numpy as jnp
from jax import lax
from jax.experimental import pallas as pl
from jax.experimental.pallas import tpu as pltpu
from jax.experimental.pallas import tpu_sc as plsc

NODE_NUM = 50000
DIM = 128
K1 = 3
M = 2
NPAIR = 2 * K1
PWD = NPAIR * NPAIR
BATCH = 4096

_NC = 2
_NS = 16
_NW = _NC * _NS
_IDS = 2 * BATCH
_PER_W = _IDS // _NW
_CH = 128
_LANES = 16


def _sc_fused_gather(ids_hbm, rp_hbm, wb_hbm, out_hbm, idx_v, idxa_v,
                     b0, b1, fb, wv, sem):
    wid = lax.axis_index("s") * _NC + lax.axis_index("c")
    base = wid * _PER_W
    half = base // BATCH
    brow = base - half * BATCH

    pltpu.sync_copy(ids_hbm.at[pl.ds(base, _PER_W)], idx_v)
    pltpu.sync_copy(wb_hbm, wv)

    for k in range(K1):
        w0 = wv[k, :]
        w1 = wv[K1 + k, :]
        for c0 in range(0, _PER_W, _CH):
            for t in range(_CH // _LANES):
                v = idx_v[pl.ds(c0 + t * _LANES, _LANES)]
                idxa_v[pl.ds(t * _LANES, _LANES)] = v + (k * NODE_NUM)
            pltpu.async_copy(rp_hbm.at[idxa_v], b0, sem).wait()
            for t in range(_CH // _LANES):
                v = idx_v[pl.ds(c0 + t * _LANES, _LANES)]
                idxa_v[pl.ds(t * _LANES, _LANES)] = v + ((K1 + k) * NODE_NUM)
            pltpu.async_copy(rp_hbm.at[idxa_v], b1, sem).wait()

            def fuse_row(c, _):
                for l in range(DIM // _LANES):
                    sl = pl.ds(l * _LANES, _LANES)
                    fb[c, sl] = b0[c, sl] * w0 + b1[c, sl] * w1
                return 0

            lax.fori_loop(0, _CH, fuse_row, 0)
            pltpu.sync_copy(
                fb, out_hbm.at[half * K1 + k, pl.ds(brow + c0, _CH)])


_sc_gather_call = pl.kernel(
    _sc_fused_gather,
    out_type=jax.ShapeDtypeStruct((NPAIR, BATCH, DIM), jnp.float32),
    mesh=plsc.VectorSubcoreMesh(core_axis_name="c", subcore_axis_name="s"),
    scratch_types=[
        pltpu.VMEM((_PER_W,), jnp.int32),
        pltpu.VMEM((_CH,), jnp.int32),
        pltpu.VMEM((_CH, DIM), jnp.float32),
        pltpu.VMEM((_CH, DIM), jnp.float32),
        pltpu.VMEM((_CH, DIM), jnp.float32),
        pltpu.VMEM((NPAIR, _LANES), jnp.float32),
        pltpu.SemaphoreType.DMA,
    ],
)


_BBLK = 512


def _tc_gram_mlp(rp_ref, w1_ref, b1_ref, w2_ref, b2_ref, out_ref):
    rows = [rp_ref[i, :, :] for i in range(NPAIR)]
    ent = {}
    for i in range(NPAIR):
        for j in range(i, NPAIR):
            ent[(i, j)] = jnp.sum(rows[i] * rows[j], axis=1, keepdims=True)
    cols = []
    for i in range(NPAIR):
        for j in range(NPAIR):
            cols.append(ent[(i, j)] if i <= j else ent[(j, i)])
    feat = jnp.concatenate(cols, axis=1)
    feat = jnp.where(feat < 0.0, 0.0, feat)
    feat = jnp.log(feat + 1.0)
    h = jnp.dot(feat, w1_ref[...], preferred_element_type=jnp.float32)
    h = jnp.maximum(h + b1_ref[...], 0.0)
    out_ref[...] = (
        jnp.dot(h, w2_ref[...], preferred_element_type=jnp.float32)
        + b2_ref[...])


def kernel(src_node_ids, dst_node_ids, RP, lambda_weights, W1, b1, W2, b2):
    Wsm = jax.nn.softmax(lambda_weights, axis=1)
    wb = jnp.broadcast_to(
        jnp.transpose(Wsm).reshape(NPAIR, 1), (NPAIR, _LANES)
    ).astype(jnp.float32)
    ids = jnp.concatenate(
        [src_node_ids, dst_node_ids]).astype(jnp.int32)
    rp_flat = RP.reshape(M * K1 * NODE_NUM, DIM)

    fused = _sc_gather_call(ids, rp_flat, wb)

    nblk = BATCH // _BBLK
    out = pl.pallas_call(
        _tc_gram_mlp,
        grid=(nblk,),
        in_specs=[
            pl.BlockSpec((NPAIR, _BBLK, DIM), lambda i: (0, i, 0)),
            pl.BlockSpec((PWD, 4 * PWD), lambda i: (0, 0)),
            pl.BlockSpec((1, 4 * PWD), lambda i: (0, 0)),
            pl.BlockSpec((4 * PWD, PWD), lambda i: (0, 0)),
            pl.BlockSpec((1, PWD), lambda i: (0, 0)),
        ],
        out_specs=pl.BlockSpec((_BBLK, PWD), lambda i: (i, 0)),
        out_shape=jax.ShapeDtypeStruct((BATCH, PWD), jnp.float32),
    )(fused, W1, b1.reshape(1, 4 * PWD), W2, b2.reshape(1, PWD))
    return out
